# baseline (speedup 1.0000x reference)
import functools

import jax
import jax.numpy as jnp
from jax.experimental import pallas as pl
from jax.experimental.pallas import tpu as pltpu


def _moa_body(x_ref, wall_ref, ball_ref, va_ref, ua_ref, wa_ref,
              wf_ref, bfr_ref, w2_ref, b2_ref, wh_ref, bh_ref,
              lat_out, log_out, g_out,
              rsum, psum, zsum,
              *, n_total, blk_n, n_b, n_nb, n_e):
    b = pl.program_id(0)
    nb = pl.program_id(1)

    @pl.when(jnp.logical_and(b == 0, nb == 0))
    def _init():
        rsum[...] = jnp.zeros_like(rsum)
        psum[...] = jnp.zeros_like(psum)
        zsum[...] = jnp.zeros_like(zsum)

    xb = x_ref[0].astype(jnp.bfloat16)

    h0 = jax.nn.relu(
        jax.lax.dot(xb, wall_ref[0], preferred_element_type=jnp.float32)
        + ball_ref[0][None, :])
    rsum[pl.ds(b, 1), :] += jnp.sum(h0, axis=0, keepdims=True)

    for e in range(n_e):
        h = jax.nn.relu(
            jax.lax.dot(xb, wall_ref[e + 1], preferred_element_type=jnp.float32)
            + ball_ref[e + 1][None, :])
        hb = h.astype(jnp.bfloat16)
        t = jnp.tanh(jax.lax.dot(hb, va_ref[e],
                                 preferred_element_type=jnp.float32))
        u = jax.nn.sigmoid(jax.lax.dot(hb, ua_ref[e],
                                       preferred_element_type=jnp.float32))
        a = t * u
        s = jnp.sum(a * wa_ref[e][None, :], axis=1, keepdims=True)
        w = jnp.exp(s)
        r = e * n_b + b
        psum[pl.ds(r, 1), :] += jnp.sum(h * w, axis=0, keepdims=True)
        zsum[pl.ds(r, 1), :] += jnp.full((1, 128), jnp.sum(w), jnp.float32)

    @pl.when(jnp.logical_and(b == n_b - 1, nb == n_nb - 1))
    def _tail():
        rin = rsum[...] / jnp.float32(n_total)
        glog = jax.lax.dot(rin, wf_ref[...],
                           preferred_element_type=jnp.float32) + bfr_ref[...]
        glog = glog - jnp.max(glog, axis=-1, keepdims=True)
        gexp = jnp.exp(glog)
        g = gexp / jnp.sum(gexp, axis=-1, keepdims=True)
        g_out[...] = g

        pooled = psum[...] / zsum[:, 0:1]
        lat_acc = jnp.zeros(lat_out.shape, jnp.float32)
        log_acc = jnp.zeros(log_out.shape, jnp.float32)
        for e in range(n_e):
            pe = pooled[e * n_b:(e + 1) * n_b, :]
            lat_e = jax.nn.relu(
                jax.lax.dot(pe, w2_ref[e],
                            preferred_element_type=jnp.float32)
                + b2_ref[e][None, :])
            log_e = jax.lax.dot(lat_e, wh_ref[e],
                                preferred_element_type=jnp.float32) \
                + bh_ref[e][None, :]
            ge = g[:, e:e + 1]
            lat_acc = lat_acc + ge * lat_e
            log_acc = log_acc + ge * log_e
        lat_out[...] = lat_acc
        log_out[...] = log_acc


@jax.jit
def kernel(x, Wr, br, Wf, bf, W1, b1, Va, Ua, wa, W2, b2, Wh, bh):
    B, N, D = x.shape
    E, _, H = W1.shape
    LAT = W2.shape[-1]
    C = Wh.shape[-1]

    blk_n = 512 if N % 512 == 0 else N
    n_nb = N // blk_n

    wall = jnp.concatenate([Wr[None], W1], axis=0).astype(jnp.bfloat16)
    ball = jnp.concatenate([br[None], b1], axis=0)
    vab = Va.astype(jnp.bfloat16)
    uab = Ua.astype(jnp.bfloat16)
    wa2 = wa[..., 0]

    body = functools.partial(_moa_body, n_total=N, blk_n=blk_n,
                             n_b=B, n_nb=n_nb, n_e=E)

    full = lambda arr: pl.BlockSpec(arr.shape, lambda b, nb: (0,) * arr.ndim)

    lat, log, g = pl.pallas_call(
        body,
        grid=(B, n_nb),
        in_specs=[
            pl.BlockSpec((1, blk_n, D), lambda b, nb: (b, nb, 0)),
            full(wall), full(ball), full(vab), full(uab), full(wa2),
            full(Wf), full(bf), full(W2), full(b2), full(Wh), full(bh),
        ],
        out_specs=[
            pl.BlockSpec((B, LAT), lambda b, nb: (0, 0)),
            pl.BlockSpec((B, C), lambda b, nb: (0, 0)),
            pl.BlockSpec((B, E), lambda b, nb: (0, 0)),
        ],
        out_shape=[
            jax.ShapeDtypeStruct((B, LAT), jnp.float32),
            jax.ShapeDtypeStruct((B, C), jnp.float32),
            jax.ShapeDtypeStruct((B, E), jnp.float32),
        ],
        scratch_shapes=[
            pltpu.VMEM((B, H), jnp.float32),
            pltpu.VMEM((E * B, H), jnp.float32),
            pltpu.VMEM((E * B, 128), jnp.float32),
        ],
        compiler_params=pltpu.CompilerParams(
            dimension_semantics=("arbitrary", "arbitrary"),
            allow_input_fusion=[False, True, True, True, True, True,
                                False, False, False, False, False, False],
        ),
    )(x, wall, ball, vab, uab, wa2, Wf, bf, W2, b2, Wh, bh)

    return (lat, log, g)


# speedup vs baseline: 1.0602x; 1.0602x over previous
import functools

import jax
import jax.numpy as jnp
from jax.experimental import pallas as pl
from jax.experimental.pallas import tpu as pltpu


def _moa_body(x_ref, wall_ref, ball_ref, va_ref, ua_ref, wa_ref,
              wf_ref, bfr_ref, w2_ref, b2_ref, wh_ref, bh_ref,
              lat_out, log_out, g_out,
              vaub, rsum, psum, zsum,
              *, n_total, blk_n, n_b, n_nb, n_e):
    b = pl.program_id(0)
    nb = pl.program_id(1)
    n_h = psum.shape[-1]
    n_a = wa_ref.shape[-1]

    @pl.when(jnp.logical_and(b == 0, nb == 0))
    def _init():
        for k in range(n_e):
            vaub[k, :, :n_a] = va_ref[k].astype(jnp.bfloat16)
            vaub[k, :, n_a:] = ua_ref[k].astype(jnp.bfloat16)
        rsum[...] = jnp.zeros_like(rsum)
        psum[...] = jnp.zeros_like(psum)
        zsum[...] = jnp.zeros_like(zsum)

    xb = x_ref[0].astype(jnp.bfloat16)

    hbcat = jax.nn.relu(
        (jax.lax.dot(xb, wall_ref[...], preferred_element_type=jnp.float32)
         + ball_ref[...]).astype(jnp.bfloat16))

    rsum[pl.ds(b, 1), :] += jnp.sum(
        hbcat[:, :n_h].astype(jnp.float32), axis=0, keepdims=True)

    for e in range(n_e):
        hb = hbcat[:, (e + 1) * n_h:(e + 2) * n_h]
        tu = jax.lax.dot(hb, vaub[e], preferred_element_type=jnp.float32)
        a = jnp.tanh(tu[:, :n_a]) * jax.nn.sigmoid(tu[:, n_a:])
        s = jnp.sum(a * wa_ref[e][None, :], axis=1, keepdims=True)
        w = jnp.exp(s)
        r = e * n_b + b
        psum[pl.ds(r, 1), :] += jnp.sum(
            hb.astype(jnp.float32) * w, axis=0, keepdims=True)
        zsum[pl.ds(r, 1), :] += jnp.full((1, 128), jnp.sum(w), jnp.float32)

    @pl.when(jnp.logical_and(b == n_b - 1, nb == n_nb - 1))
    def _tail():
        rin = rsum[...] / jnp.float32(n_total)
        glog = jax.lax.dot(rin, wf_ref[...],
                           preferred_element_type=jnp.float32) + bfr_ref[...]
        glog = glog - jnp.max(glog, axis=-1, keepdims=True)
        gexp = jnp.exp(glog)
        g = gexp / jnp.sum(gexp, axis=-1, keepdims=True)
        g_out[...] = g

        pooled = psum[...] / zsum[:, 0:1]
        lat_acc = jnp.zeros(lat_out.shape, jnp.float32)
        log_acc = jnp.zeros(log_out.shape, jnp.float32)
        for e in range(n_e):
            pe = pooled[e * n_b:(e + 1) * n_b, :]
            lat_e = jax.nn.relu(
                jax.lax.dot(pe, w2_ref[e],
                            preferred_element_type=jnp.float32)
                + b2_ref[e][None, :])
            log_e = jax.lax.dot(lat_e, wh_ref[e],
                                preferred_element_type=jnp.float32) \
                + bh_ref[e][None, :]
            ge = g[:, e:e + 1]
            lat_acc = lat_acc + ge * lat_e
            log_acc = log_acc + ge * log_e
        lat_out[...] = lat_acc
        log_out[...] = log_acc


@jax.jit
def kernel(x, Wr, br, Wf, bf, W1, b1, Va, Ua, wa, W2, b2, Wh, bh):
    B, N, D = x.shape
    E, _, H = W1.shape
    A = Va.shape[-1]
    LAT = W2.shape[-1]
    C = Wh.shape[-1]

    blk_n = 512 if N % 512 == 0 else N
    n_nb = N // blk_n

    wall = jnp.concatenate(
        [Wr] + [W1[e] for e in range(E)], axis=1).astype(jnp.bfloat16)
    ball = jnp.concatenate([br] + [b1[e] for e in range(E)])[None, :]
    wa2 = wa[..., 0]

    body = functools.partial(_moa_body, n_total=N, blk_n=blk_n,
                             n_b=B, n_nb=n_nb, n_e=E)

    full = lambda arr: pl.BlockSpec(arr.shape, lambda b, nb: (0,) * arr.ndim)

    lat, log, g = pl.pallas_call(
        body,
        grid=(B, n_nb),
        in_specs=[
            pl.BlockSpec((1, blk_n, D), lambda b, nb: (b, nb, 0)),
            full(wall), full(ball), full(Va), full(Ua), full(wa2),
            full(Wf), full(bf), full(W2), full(b2), full(Wh), full(bh),
        ],
        out_specs=[
            pl.BlockSpec((B, LAT), lambda b, nb: (0, 0)),
            pl.BlockSpec((B, C), lambda b, nb: (0, 0)),
            pl.BlockSpec((B, E), lambda b, nb: (0, 0)),
        ],
        out_shape=[
            jax.ShapeDtypeStruct((B, LAT), jnp.float32),
            jax.ShapeDtypeStruct((B, C), jnp.float32),
            jax.ShapeDtypeStruct((B, E), jnp.float32),
        ],
        scratch_shapes=[
            pltpu.VMEM((E, H, 2 * A), jnp.bfloat16),
            pltpu.VMEM((B, H), jnp.float32),
            pltpu.VMEM((E * B, H), jnp.float32),
            pltpu.VMEM((E * B, 128), jnp.float32),
        ],
        compiler_params=pltpu.CompilerParams(
            dimension_semantics=("arbitrary", "arbitrary"),
        ),
    )(x, wall, ball, Va, Ua, wa2, Wf, bf, W2, b2, Wh, bh)

    return (lat, log, g)
